# baseline (device time: 49800 ns/iter reference)
import jax
import jax.numpy as jnp
from jax import lax
from jax.experimental import pallas as pl
from jax.experimental.pallas import tpu as pltpu

B, S, H, Dh, Dr = 2, 256, 16, 64, 32
D = 1024
DC_SH = 64
NCHUNK = 2
S_CHK = S // NCHUNK


def _dot(a, b, trans_b=False):
    dn = (((1,), (1 if trans_b else 0,)), ((), ()))
    return lax.dot_general(a, b, dn, preferred_element_type=jnp.float32)


def kernel(x, Wdkv, Wuk, Wuv, Wq, Wqr, Wkr, Wo):
    def body(x_ref, wdkv_ref, wuk_ref, wuv_ref, wq_ref, wqr_ref, wkr_ref,
             wo_ref, out_ref, c_snd, c_rcv, wuk_r, wuv_r,
             send_sems, recv_sems):
        my_x = lax.axis_index("x")
        my_y = lax.axis_index("y")
        y_nbr = (my_x, 1 - my_y)
        x_nbr = (1 - my_x, my_y)

        barrier = pltpu.get_barrier_semaphore()
        pl.semaphore_signal(barrier, inc=1, device_id=y_nbr,
                            device_id_type=pl.DeviceIdType.MESH)
        pl.semaphore_signal(barrier, inc=1, device_id=x_nbr,
                            device_id_type=pl.DeviceIdType.MESH)
        pl.semaphore_wait(barrier, 2)

        xb = x_ref[pl.ds(my_x, 1)].reshape(S, D)
        c1 = _dot(xb, wdkv_ref[...])
        c_snd[...] = c1
        rdmas = []
        for i, (src, dst) in enumerate(
                [(c_snd, c_rcv), (wuk_ref, wuk_r), (wuv_ref, wuv_r)]):
            r = pltpu.make_async_remote_copy(
                src_ref=src, dst_ref=dst,
                send_sem=send_sems.at[i], recv_sem=recv_sems.at[i],
                device_id=y_nbr, device_id_type=pl.DeviceIdType.MESH)
            r.start()
            rdmas.append(r)

        Q = _dot(xb, wq_ref[...])
        Qr = _dot(xb, wqr_ref[...])
        Kr = _dot(xb, wkr_ref[...])
        K = _dot(c1, wuk_ref[...])
        V = _dot(c1, wuv_ref[...])
        scale = (Dh + Dr) ** -0.5
        s_rope = [_dot(Qr[:, h * Dr:(h + 1) * Dr], Kr, trans_b=True)
                  for h in range(H)]

        rdmas[0].wait()
        rdmas[1].wait()
        c2 = c_rcv[...]
        K = K + _dot(c2, wuk_r[...])
        rdmas[2].wait()
        V = V + _dot(c2, wuv_r[...])

        out_rdmas = []
        for ci in range(NCHUNK):
            rows = slice(ci * S_CHK, (ci + 1) * S_CHK)
            o_parts = []
            for h in range(H):
                qh = Q[rows, h * Dh:(h + 1) * Dh]
                kh = K[:, h * Dh:(h + 1) * Dh]
                vh = V[:, h * Dh:(h + 1) * Dh]
                s = (_dot(qh, kh, trans_b=True) + s_rope[h][rows]) * scale
                m = jnp.max(s, axis=-1, keepdims=True)
                p = jnp.exp(s - m)
                p = p / jnp.sum(p, axis=-1, keepdims=True)
                o_parts.append(_dot(p, vh))
            o_chunk = jnp.concatenate(o_parts, axis=-1)
            out_chunk = _dot(o_chunk, wo_ref[...])
            idx = (pl.ds(my_x, 1), pl.ds(ci * S_CHK, S_CHK))
            out_ref[idx] = out_chunk[None]
            r = pltpu.make_async_remote_copy(
                src_ref=out_ref.at[idx], dst_ref=out_ref.at[idx],
                send_sem=send_sems.at[3 + ci], recv_sem=recv_sems.at[3 + ci],
                device_id=x_nbr, device_id_type=pl.DeviceIdType.MESH)
            r.start()
            out_rdmas.append(r)

        for r in out_rdmas:
            r.wait()

    return pl.pallas_call(
        body,
        out_shape=jax.ShapeDtypeStruct((B, S, D), jnp.float32),
        in_specs=[pl.BlockSpec(memory_space=pltpu.VMEM)] * 8,
        out_specs=pl.BlockSpec(memory_space=pltpu.VMEM),
        scratch_shapes=[
            pltpu.VMEM((S, DC_SH), jnp.float32),
            pltpu.VMEM((S, DC_SH), jnp.float32),
            pltpu.VMEM((DC_SH, D), jnp.float32),
            pltpu.VMEM((DC_SH, D), jnp.float32),
            pltpu.SemaphoreType.DMA((3 + NCHUNK,)),
            pltpu.SemaphoreType.DMA((3 + NCHUNK,)),
        ],
        compiler_params=pltpu.CompilerParams(collective_id=0),
    )(x, Wdkv, Wuk, Wuv, Wq, Wqr, Wkr, Wo)
